# Initial kernel scaffold; baseline (speedup 1.0000x reference)
import jax
import jax.numpy as jnp
from jax.experimental import pallas as pl


def kernel(x, one_hop_neighbors, treat_binary, ehat, b, W1, b1, W2, b2, W3, b3):
    raise NotImplementedError("write your pallas kernel here")


# speedup vs baseline: 4.0501x; 4.0501x over previous
import functools

import jax
import jax.numpy as jnp
from jax import lax
from jax.experimental import pallas as pl
from jax.experimental.pallas import tpu as pltpu
from jax.experimental.pallas import tpu_sc as plsc

NC, NS = 2, 16
NW = NC * NS


def _mlp_pre_kernel(x_ref, w1at_ref, w1bt_ref, b1_ref, t_ref, e_ref,
                    a_ref, u_ref, te_ref):
    xb = x_ref[...]
    a_ref[...] = jnp.dot(xb, w1at_ref[...],
                         preferred_element_type=jnp.float32,
                         precision=lax.Precision.HIGHEST) + b1_ref[...]
    u_ref[...] = jnp.dot(xb, w1bt_ref[...],
                         preferred_element_type=jnp.float32,
                         precision=lax.Precision.HIGHEST)
    te_ref[...] = t_ref[...] - e_ref[...]


def _mlp_main_kernel(br, k, ug_ref, ag_ref, w2t_ref, b2_ref, w3_ref, b3_ref,
                     out_ref):
    bp = br * k
    h = ag_ref[...]
    h = jnp.broadcast_to(h[:, None, :], (br, k, h.shape[-1]))
    h = jnp.reshape(h, (bp, h.shape[-1]))
    h1 = jnp.maximum(h + ug_ref[...], 0.0)
    h2 = jnp.dot(h1, w2t_ref[...],
                 preferred_element_type=jnp.float32,
                 precision=lax.Precision.HIGHEST) + b2_ref[...]
    h2 = jnp.maximum(h2, 0.0)
    mlp = jnp.sum(h2 * w3_ref[...], axis=1, keepdims=True) + b3_ref[0, 0]
    out_ref[...] = mlp


def _softmax_kernel(mlp_ref, te_ref, b_ref, vals_ref, yp_ref):
    m = mlp_ref[...]
    am = b_ref[0, 0] * jnp.abs(m)
    mx = jnp.max(am, axis=1, keepdims=True)
    e = jnp.exp(am - mx)
    s = jnp.sum(e, axis=1, keepdims=True)
    v = m * (e / s)
    vals_ref[...] = v
    yp_ref[...] = jnp.sum(te_ref[...] * v, axis=1, keepdims=True)


def kernel(x, one_hop_neighbors, treat_binary, ehat, b, W1, b1, W2, b2, W3, b3):
    n, d = x.shape
    R, kp1 = one_hop_neighbors.shape
    K = kp1 - 1
    H = W1.shape[0]
    P = R * K

    cur = one_hop_neighbors[:, 0]
    chosen = one_hop_neighbors[:, 1:]
    chosen_flat = chosen.reshape(-1)

    W1aT = W1[:, :d].T
    W1bT = W1[:, d:].T
    W2T = W2.T
    b1r = b1.reshape(1, H)
    b2r = b2.reshape(1, H)
    w3r = W3.reshape(1, H)
    b3r = b3.reshape(1, 1)
    br_ = jnp.asarray(b, jnp.float32).reshape(1, 1)

    BR1 = 2000
    a_all, u_all, te_all = pl.pallas_call(
        _mlp_pre_kernel,
        grid=(n // BR1,),
        in_specs=[
            pl.BlockSpec((BR1, d), lambda i: (i, 0)),
            pl.BlockSpec((d, H), lambda i: (0, 0)),
            pl.BlockSpec((d, H), lambda i: (0, 0)),
            pl.BlockSpec((1, H), lambda i: (0, 0)),
            pl.BlockSpec((BR1, 1), lambda i: (i, 0)),
            pl.BlockSpec((BR1, 1), lambda i: (i, 0)),
        ],
        out_specs=[
            pl.BlockSpec((BR1, H), lambda i: (i, 0)),
            pl.BlockSpec((BR1, H), lambda i: (i, 0)),
            pl.BlockSpec((BR1, 1), lambda i: (i, 0)),
        ],
        out_shape=[
            jax.ShapeDtypeStruct((n, H), jnp.float32),
            jax.ShapeDtypeStruct((n, H), jnp.float32),
            jax.ShapeDtypeStruct((n, 1), jnp.float32),
        ],
    )(x, W1aT, W1bT, b1r, treat_binary.reshape(n, 1), ehat.reshape(n, 1))
    te_flat = te_all.reshape(n)

    PPW = P // NW
    CPW = -(-R // (NW * 8)) * 8
    RP = CPW * NW
    cur_pad = jnp.concatenate([cur, jnp.zeros((RP - R,), jnp.int32)])

    CH = 400
    n_full, rem = PPW // CH, PPW % CH

    mesh = plsc.VectorSubcoreMesh(core_axis_name="c", subcore_axis_name="s")

    @functools.partial(
        pl.kernel, mesh=mesh,
        out_type=[
            jax.ShapeDtypeStruct((RP, H), jnp.float32),
            jax.ShapeDtypeStruct((P, H), jnp.float32),
            jax.ShapeDtypeStruct((P,), jnp.float32),
        ],
        scratch_types=[
            pltpu.VMEM((PPW,), jnp.int32),
            pltpu.VMEM((CPW,), jnp.int32),
            pltpu.VMEM((CH, H), jnp.float32),
            pltpu.VMEM((CPW, H), jnp.float32),
            pltpu.VMEM((PPW,), jnp.float32),
            pltpu.SemaphoreType.DMA,
        ],
    )
    def gather_k(a_hbm, u_hbm, te_hbm, curp_hbm, cf_hbm,
                 ag_hbm, ug_hbm, teg_hbm,
                 cidx_v, curidx_v, rows_v, arows_v, teb_v, sem):
        wid = lax.axis_index("s") * NC + lax.axis_index("c")
        pbase = wid * PPW
        pltpu.sync_copy(cf_hbm.at[pl.ds(pbase, PPW)], cidx_v)
        pltpu.async_copy(te_hbm.at[cidx_v], teb_v, sem).wait()
        pltpu.sync_copy(teb_v, teg_hbm.at[pl.ds(pbase, PPW)])
        for c in range(n_full + (1 if rem else 0)):
            off = c * CH
            ch = CH if c < n_full else rem
            pltpu.async_copy(u_hbm.at[cidx_v.at[pl.ds(off, ch)]],
                             rows_v.at[pl.ds(0, ch)], sem).wait()
            pltpu.sync_copy(rows_v.at[pl.ds(0, ch)],
                            ug_hbm.at[pl.ds(pbase + off, ch)])
        cbase = wid * CPW
        pltpu.sync_copy(curp_hbm.at[pl.ds(cbase, CPW)], curidx_v)
        pltpu.async_copy(a_hbm.at[curidx_v], arows_v, sem).wait()
        pltpu.sync_copy(arows_v, ag_hbm.at[pl.ds(cbase, CPW)])

    ag_pad, ug, teg = gather_k(a_all, u_all, te_flat, cur_pad, chosen_flat)

    BR3 = 400
    BP3 = BR3 * K
    mlp_flat = pl.pallas_call(
        functools.partial(_mlp_main_kernel, BR3, K),
        grid=(P // BP3,),
        in_specs=[
            pl.BlockSpec((BP3, H), lambda i: (i, 0)),
            pl.BlockSpec((BR3, H), lambda i: (i, 0)),
            pl.BlockSpec((H, H), lambda i: (0, 0)),
            pl.BlockSpec((1, H), lambda i: (0, 0)),
            pl.BlockSpec((1, H), lambda i: (0, 0)),
            pl.BlockSpec((1, 1), lambda i: (0, 0), memory_space=pltpu.SMEM),
        ],
        out_specs=pl.BlockSpec((BP3, 1), lambda i: (i, 0)),
        out_shape=jax.ShapeDtypeStruct((P, 1), jnp.float32),
    )(ug, ag_pad, W2T, b2r, w3r, b3r)

    BR4 = 1000
    vals, ypred = pl.pallas_call(
        _softmax_kernel,
        grid=(R // BR4,),
        in_specs=[
            pl.BlockSpec((BR4, K), lambda i: (i, 0)),
            pl.BlockSpec((BR4, K), lambda i: (i, 0)),
            pl.BlockSpec((1, 1), lambda i: (0, 0), memory_space=pltpu.SMEM),
        ],
        out_specs=[
            pl.BlockSpec((BR4, K), lambda i: (i, 0)),
            pl.BlockSpec((BR4, 1), lambda i: (i, 0)),
        ],
        out_shape=[
            jax.ShapeDtypeStruct((R, K), jnp.float32),
            jax.ShapeDtypeStruct((R, 1), jnp.float32),
        ],
    )(mlp_flat.reshape(R, K), teg.reshape(R, K), br_)
    Y_pred = ypred.reshape(R)

    idx_pairs = (cur[:, None] * n + chosen).reshape(-1)
    idx_diag = cur * (n + 1)
    U = P + R
    CH2 = 8192
    NCH2 = -(-U // CH2)
    UP = NCH2 * CH2
    pad = UP - U
    idx_all = jnp.concatenate(
        [idx_pairs, idx_diag,
         jnp.broadcast_to(idx_diag[-1:], (pad,))])
    vals_all = jnp.concatenate(
        [vals.reshape(-1), jnp.zeros((R + pad,), jnp.float32)])

    @functools.partial(
        pl.kernel, mesh=mesh, out_type=[],
        scratch_types=[
            pltpu.VMEM((CH2,), jnp.int32),
            pltpu.VMEM((CH2,), jnp.float32),
            pltpu.SemaphoreType.DMA,
        ],
    )
    def scatter_k(idx_hbm, val_hbm, pw_hbm, idx_v, val_v, sem):
        wid = lax.axis_index("s") * NC + lax.axis_index("c")

        @pl.when(wid == 0)
        def _():
            for c in range(NCH2):
                off = c * CH2
                pltpu.sync_copy(idx_hbm.at[pl.ds(off, CH2)], idx_v)
                pltpu.sync_copy(val_hbm.at[pl.ds(off, CH2)], val_v)
                pltpu.async_copy(val_v, pw_hbm.at[idx_v], sem).wait()

    pw_ref = jax.new_ref(jnp.zeros((n * n,), jnp.float32))
    scatter_k(idx_all, vals_all, pw_ref)
    pairwise = pw_ref[...].reshape(n, n)

    return (Y_pred, pairwise)
